# baseline (speedup 1.0000x reference)
import jax
import jax.numpy as jnp
from jax import lax
from jax.experimental import pallas as pl
from jax.experimental.pallas import tpu as pltpu
from jax.experimental.pallas import tpu_sc as plsc

N = 10000
E = 320000
D = 128
H = 128
C = 64

NS = 16
K = 80
RPT = 624
TAIL = N - NS * RPT
SLAB = 104
NSLAB = RPT // SLAB
DEGW = 16
ROWS_BLK = 1000


def _sc_aggregate(feat, src, dst, zeros_feat):
    mesh = plsc.VectorSubcoreMesh(core_axis_name="c", subcore_axis_name="s",
                                  num_cores=1)
    epw = E // NS

    def body(feat_hbm, src_hbm, dst_hbm, zf_hbm, acc_out, src0, src1,
             dst0, dst1, rows0, rows1, stg_v, acc_sh, sg0, sg1, ss0, ss1):
        sid = lax.axis_index("s")
        row0 = sid * RPT

        pltpu.sync_copy(zf_hbm.at[pl.ds(0, SLAB)], stg_v)
        for j in range(NSLAB):
            pltpu.sync_copy(stg_v, acc_sh.at[pl.ds(row0 + j * SLAB, SLAB)])

        @pl.when(sid == NS - 1)
        def _():
            pltpu.sync_copy(stg_v.at[pl.ds(0, TAIL)],
                            acc_sh.at[pl.ds(NS * RPT, TAIL)])

        plsc.subcore_barrier()
        ebase = sid * epw

        def chunk(i, carry):
            @pl.when(i > 0)
            def _():
                pltpu.make_async_copy(rows0, acc_sh.at[dst0], ss0).wait()
                pltpu.make_async_copy(rows1, acc_sh.at[dst1], ss1).wait()

            off = ebase + i * (2 * K)
            pltpu.sync_copy(src_hbm.at[pl.ds(off, K)], src0)
            g0 = pltpu.async_copy(feat_hbm.at[src0], rows0, sg0)
            pltpu.sync_copy(src_hbm.at[pl.ds(off + K, K)], src1)
            g1 = pltpu.async_copy(feat_hbm.at[src1], rows1, sg1)
            pltpu.sync_copy(dst_hbm.at[pl.ds(off, K)], dst0)
            pltpu.sync_copy(dst_hbm.at[pl.ds(off + K, K)], dst1)
            g0.wait()
            pltpu.async_copy(rows0, acc_sh.at[dst0], ss0, add=True)
            g1.wait()
            pltpu.async_copy(rows1, acc_sh.at[dst1], ss1, add=True)
            return carry

        lax.fori_loop(0, epw // (2 * K), chunk, 0)
        pltpu.make_async_copy(rows0, acc_sh.at[dst0], ss0).wait()
        pltpu.make_async_copy(rows1, acc_sh.at[dst1], ss1).wait()
        plsc.subcore_barrier()

        for j in range(NSLAB):
            r = row0 + j * SLAB
            pltpu.sync_copy(acc_sh.at[pl.ds(r, SLAB)], stg_v)
            pltpu.sync_copy(stg_v, acc_out.at[pl.ds(r, SLAB)])

        @pl.when(sid == NS - 1)
        def _2():
            pltpu.sync_copy(acc_sh.at[pl.ds(NS * RPT, TAIL)],
                            stg_v.at[pl.ds(0, TAIL)])
            pltpu.sync_copy(stg_v.at[pl.ds(0, TAIL)],
                            acc_out.at[pl.ds(NS * RPT, TAIL)])

    return pl.kernel(
        body,
        out_type=jax.ShapeDtypeStruct((N, H), jnp.float32),
        mesh=mesh,
        scratch_types=[
            pltpu.VMEM((K,), jnp.int32),
            pltpu.VMEM((K,), jnp.int32),
            pltpu.VMEM((K,), jnp.int32),
            pltpu.VMEM((K,), jnp.int32),
            pltpu.VMEM((K, H), jnp.float32),
            pltpu.VMEM((K, H), jnp.float32),
            pltpu.VMEM((SLAB, H), jnp.float32),
            pltpu.VMEM_SHARED((N, H), jnp.float32),
            pltpu.SemaphoreType.DMA,
            pltpu.SemaphoreType.DMA,
            pltpu.SemaphoreType.DMA,
            pltpu.SemaphoreType.DMA,
        ])(feat, src, dst, zeros_feat)


def _sc_degree(dst, zeros_feat, ones_deg):
    mesh = plsc.VectorSubcoreMesh(core_axis_name="c", subcore_axis_name="s",
                                  num_cores=1)
    epw = E // NS

    def body(dst_hbm, zf_hbm, ones_hbm, deg_out, dst0, dst1, ones_v, stg_v,
             deg_sh, ss0, ss1):
        sid = lax.axis_index("s")
        row0 = sid * RPT

        pltpu.sync_copy(zf_hbm.at[pl.ds(0, SLAB)], stg_v)
        pltpu.sync_copy(ones_hbm, ones_v)
        for j in range(NSLAB):
            pltpu.sync_copy(stg_v, deg_sh.at[pl.ds(row0 + j * SLAB, SLAB)])

        @pl.when(sid == NS - 1)
        def _():
            pltpu.sync_copy(stg_v.at[pl.ds(0, TAIL)],
                            deg_sh.at[pl.ds(NS * RPT, TAIL)])

        plsc.subcore_barrier()
        ebase = sid * epw

        def chunk(i, carry):
            @pl.when(i > 0)
            def _():
                pltpu.make_async_copy(ones_v, deg_sh.at[dst0], ss0).wait()
                pltpu.make_async_copy(ones_v, deg_sh.at[dst1], ss1).wait()

            off = ebase + i * (2 * K)
            pltpu.sync_copy(dst_hbm.at[pl.ds(off, K)], dst0)
            pltpu.async_copy(ones_v, deg_sh.at[dst0], ss0, add=True)
            pltpu.sync_copy(dst_hbm.at[pl.ds(off + K, K)], dst1)
            pltpu.async_copy(ones_v, deg_sh.at[dst1], ss1, add=True)
            return carry

        lax.fori_loop(0, epw // (2 * K), chunk, 0)
        pltpu.make_async_copy(ones_v, deg_sh.at[dst0], ss0).wait()
        pltpu.make_async_copy(ones_v, deg_sh.at[dst1], ss1).wait()
        plsc.subcore_barrier()

        for j in range(NSLAB):
            r = row0 + j * SLAB
            pltpu.sync_copy(deg_sh.at[pl.ds(r, SLAB)], stg_v)
            pltpu.sync_copy(stg_v, deg_out.at[pl.ds(r, SLAB)])

        @pl.when(sid == NS - 1)
        def _2():
            pltpu.sync_copy(deg_sh.at[pl.ds(NS * RPT, TAIL)],
                            stg_v.at[pl.ds(0, TAIL)])
            pltpu.sync_copy(stg_v.at[pl.ds(0, TAIL)],
                            deg_out.at[pl.ds(NS * RPT, TAIL)])

    return pl.kernel(
        body,
        out_type=jax.ShapeDtypeStruct((N, H), jnp.float32),
        mesh=mesh,
        scratch_types=[
            pltpu.VMEM((K,), jnp.int32),
            pltpu.VMEM((K,), jnp.int32),
            pltpu.VMEM((K, H), jnp.float32),
            pltpu.VMEM((SLAB, H), jnp.float32),
            pltpu.VMEM_SHARED((N, H), jnp.float32),
            pltpu.SemaphoreType.DMA,
            pltpu.SemaphoreType.DMA,
        ])(dst, zeros_feat, ones_deg)


def _tc_lin1_body(x_ref, wl_ref, wr_ref, b_ref, p_ref, r_ref):
    xb = x_ref[...]
    p_ref[...] = jnp.dot(xb, wl_ref[...], preferred_element_type=jnp.float32)
    r_ref[...] = jnp.dot(xb, wr_ref[...],
                         preferred_element_type=jnp.float32) + b_ref[...]


def _tc_lin1(x, Wl, Wr, b):
    return pl.pallas_call(
        _tc_lin1_body,
        grid=(N // ROWS_BLK,),
        in_specs=[
            pl.BlockSpec((ROWS_BLK, D), lambda i: (i, 0)),
            pl.BlockSpec((D, H), lambda i: (0, 0)),
            pl.BlockSpec((D, H), lambda i: (0, 0)),
            pl.BlockSpec((1, H), lambda i: (0, 0)),
        ],
        out_specs=[
            pl.BlockSpec((ROWS_BLK, H), lambda i: (i, 0)),
            pl.BlockSpec((ROWS_BLK, H), lambda i: (i, 0)),
        ],
        out_shape=[
            jax.ShapeDtypeStruct((N, H), jnp.float32),
            jax.ShapeDtypeStruct((N, H), jnp.float32),
        ],
    )(x, Wl, Wr, b)


def _tc_mid_body(a_ref, d_ref, r1_ref, h_ref):
    dm = jnp.maximum(d_ref[:, 0:1], 1.0)
    h_ref[...] = jnp.maximum(a_ref[...] / dm + r1_ref[...], 0.0)


def _tc_mid(a, d, R1):
    return pl.pallas_call(
        _tc_mid_body,
        grid=(N // ROWS_BLK,),
        in_specs=[
            pl.BlockSpec((ROWS_BLK, H), lambda i: (i, 0)),
            pl.BlockSpec((ROWS_BLK, H), lambda i: (i, 0)),
            pl.BlockSpec((ROWS_BLK, H), lambda i: (i, 0)),
        ],
        out_specs=pl.BlockSpec((ROWS_BLK, H), lambda i: (i, 0)),
        out_shape=jax.ShapeDtypeStruct((N, H), jnp.float32),
    )(a, d, R1)


def _tc_out_body(a_ref, d_ref, h_ref, wl_ref, wr_ref, b_ref, o_ref):
    dm = jnp.maximum(d_ref[:, 0:1], 1.0)
    mean2 = a_ref[...] / dm
    o = (jnp.dot(mean2, wl_ref[...], preferred_element_type=jnp.float32)
         + jnp.dot(h_ref[...], wr_ref[...],
                   preferred_element_type=jnp.float32)
         + b_ref[...])
    m = jnp.max(o, axis=1, keepdims=True)
    e = jnp.exp(o - m)
    lse = jnp.log(jnp.sum(e, axis=1, keepdims=True))
    o_ref[...] = o - m - lse


def _tc_out(a, d, h, Wl2, Wr2, b2):
    return pl.pallas_call(
        _tc_out_body,
        grid=(N // ROWS_BLK,),
        in_specs=[
            pl.BlockSpec((ROWS_BLK, H), lambda i: (i, 0)),
            pl.BlockSpec((ROWS_BLK, H), lambda i: (i, 0)),
            pl.BlockSpec((ROWS_BLK, H), lambda i: (i, 0)),
            pl.BlockSpec((H, C), lambda i: (0, 0)),
            pl.BlockSpec((H, C), lambda i: (0, 0)),
            pl.BlockSpec((1, C), lambda i: (0, 0)),
        ],
        out_specs=pl.BlockSpec((ROWS_BLK, C), lambda i: (i, 0)),
        out_shape=jax.ShapeDtypeStruct((N, C), jnp.float32),
    )(a, d, h, Wl2, Wr2, b2)


@jax.jit
def kernel(x, edge_index, W_l1, W_r1, b1, W_l2, W_r2, b2):
    src = edge_index[0]
    dst = edge_index[1]
    zeros_h = jnp.zeros((N, H), jnp.float32)
    ones_deg = jnp.ones((K, H), jnp.float32)

    P1, R1 = _tc_lin1(x, W_l1, W_r1, b1.reshape(1, H))
    deg = _sc_degree(dst, zeros_h, ones_deg)
    deg, P1 = lax.optimization_barrier((deg, P1))
    acc1 = _sc_aggregate(P1, src, dst, zeros_h)
    h = _tc_mid(acc1, deg, R1)
    acc2 = _sc_aggregate(h, src, dst, zeros_h)
    return _tc_out(acc2, deg, h, W_l2, W_r2, b2.reshape(1, C))


# speedup vs baseline: 1.0997x; 1.0997x over previous
import jax
import jax.numpy as jnp
from jax import lax
from jax.experimental import pallas as pl
from jax.experimental.pallas import tpu as pltpu
from jax.experimental.pallas import tpu_sc as plsc

N = 10000
E = 320000
D = 128
H = 128
C = 64

NS = 16
K = 80
RPT = 624
TAIL = N - NS * RPT
SLAB = 48
NSLAB = RPT // SLAB
DEGW = 16
ROWS_BLK = 1000


def _sc_aggregate(feat, src, dst, zeros_feat):
    mesh = plsc.VectorSubcoreMesh(core_axis_name="c", subcore_axis_name="s",
                                  num_cores=1)
    epw = E // NS

    def body(feat_hbm, src_hbm, dst_hbm, zf_hbm, acc_out, src0, src1, src2,
             dst0, dst1, dst2, rows0, rows1, rows2, stg_v, acc_sh,
             sg0, sg1, sg2, ss0, ss1, ss2):
        sid = lax.axis_index("s")
        row0 = sid * RPT

        pltpu.sync_copy(zf_hbm.at[pl.ds(0, SLAB)], stg_v)
        for j in range(NSLAB):
            pltpu.sync_copy(stg_v, acc_sh.at[pl.ds(row0 + j * SLAB, SLAB)])

        @pl.when(sid == NS - 1)
        def _():
            pltpu.sync_copy(stg_v.at[pl.ds(0, TAIL)],
                            acc_sh.at[pl.ds(NS * RPT, TAIL)])

        plsc.subcore_barrier()
        ebase = sid * epw
        srcs = (src0, src1, src2)
        dsts = (dst0, dst1, dst2)
        rows = (rows0, rows1, rows2)
        sgs = (sg0, sg1, sg2)
        sss = (ss0, ss1, ss2)

        def chunk(i, carry):
            off = ebase + i * (3 * K)
            for b in range(3):
                pltpu.sync_copy(src_hbm.at[pl.ds(off + b * K, K)], srcs[b])

                @pl.when(i > 0)
                def _():
                    pltpu.make_async_copy(rows[b], acc_sh.at[dsts[b]],
                                          sss[b]).wait()

                pltpu.async_copy(feat_hbm.at[srcs[b]], rows[b], sgs[b])
                pltpu.sync_copy(dst_hbm.at[pl.ds(off + b * K, K)], dsts[b])
            for b in range(3):
                pltpu.make_async_copy(feat_hbm.at[srcs[b]], rows[b],
                                      sgs[b]).wait()
                pltpu.async_copy(rows[b], acc_sh.at[dsts[b]], sss[b],
                                 add=True)
            return carry

        nsteps = epw // (3 * K)
        lax.fori_loop(0, nsteps, chunk, 0)
        for b in range(3):
            pltpu.make_async_copy(rows[b], acc_sh.at[dsts[b]], sss[b]).wait()
        for r in range(epw // K - 3 * nsteps):
            off = ebase + nsteps * 3 * K + r * K
            pltpu.sync_copy(src_hbm.at[pl.ds(off, K)], src0)
            pltpu.sync_copy(dst_hbm.at[pl.ds(off, K)], dst0)
            pltpu.async_copy(feat_hbm.at[src0], rows0, sg0).wait()
            pltpu.async_copy(rows0, acc_sh.at[dst0], ss0, add=True).wait()
        plsc.subcore_barrier()

        for j in range(NSLAB):
            r = row0 + j * SLAB
            pltpu.sync_copy(acc_sh.at[pl.ds(r, SLAB)], stg_v)
            pltpu.sync_copy(stg_v, acc_out.at[pl.ds(r, SLAB)])

        @pl.when(sid == NS - 1)
        def _2():
            pltpu.sync_copy(acc_sh.at[pl.ds(NS * RPT, TAIL)],
                            stg_v.at[pl.ds(0, TAIL)])
            pltpu.sync_copy(stg_v.at[pl.ds(0, TAIL)],
                            acc_out.at[pl.ds(NS * RPT, TAIL)])

    return pl.kernel(
        body,
        out_type=jax.ShapeDtypeStruct((N, H), jnp.float32),
        mesh=mesh,
        scratch_types=(
            [pltpu.VMEM((K,), jnp.int32)] * 6
            + [pltpu.VMEM((K, H), jnp.float32)] * 3
            + [pltpu.VMEM((SLAB, H), jnp.float32),
               pltpu.VMEM_SHARED((N, H), jnp.float32)]
            + [pltpu.SemaphoreType.DMA] * 6
        ))(feat, src, dst, zeros_feat)


def _sc_degree(dst, zeros_feat, ones_deg):
    mesh = plsc.VectorSubcoreMesh(core_axis_name="c", subcore_axis_name="s",
                                  num_cores=1)
    epw = E // NS

    def body(dst_hbm, zf_hbm, ones_hbm, deg_out, dst0, dst1, ones_v, stg_v,
             deg_sh, ss0, ss1):
        sid = lax.axis_index("s")
        row0 = sid * RPT

        pltpu.sync_copy(zf_hbm.at[pl.ds(0, SLAB)], stg_v)
        pltpu.sync_copy(ones_hbm, ones_v)
        for j in range(NSLAB):
            pltpu.sync_copy(stg_v, deg_sh.at[pl.ds(row0 + j * SLAB, SLAB)])

        @pl.when(sid == NS - 1)
        def _():
            pltpu.sync_copy(stg_v.at[pl.ds(0, TAIL)],
                            deg_sh.at[pl.ds(NS * RPT, TAIL)])

        plsc.subcore_barrier()
        ebase = sid * epw

        def chunk(i, carry):
            @pl.when(i > 0)
            def _():
                pltpu.make_async_copy(ones_v, deg_sh.at[dst0], ss0).wait()
                pltpu.make_async_copy(ones_v, deg_sh.at[dst1], ss1).wait()

            off = ebase + i * (2 * K)
            pltpu.sync_copy(dst_hbm.at[pl.ds(off, K)], dst0)
            pltpu.async_copy(ones_v, deg_sh.at[dst0], ss0, add=True)
            pltpu.sync_copy(dst_hbm.at[pl.ds(off + K, K)], dst1)
            pltpu.async_copy(ones_v, deg_sh.at[dst1], ss1, add=True)
            return carry

        lax.fori_loop(0, epw // (2 * K), chunk, 0)
        pltpu.make_async_copy(ones_v, deg_sh.at[dst0], ss0).wait()
        pltpu.make_async_copy(ones_v, deg_sh.at[dst1], ss1).wait()
        plsc.subcore_barrier()

        for j in range(NSLAB):
            r = row0 + j * SLAB
            pltpu.sync_copy(deg_sh.at[pl.ds(r, SLAB)], stg_v)
            pltpu.sync_copy(stg_v, deg_out.at[pl.ds(r, SLAB)])

        @pl.when(sid == NS - 1)
        def _2():
            pltpu.sync_copy(deg_sh.at[pl.ds(NS * RPT, TAIL)],
                            stg_v.at[pl.ds(0, TAIL)])
            pltpu.sync_copy(stg_v.at[pl.ds(0, TAIL)],
                            deg_out.at[pl.ds(NS * RPT, TAIL)])

    return pl.kernel(
        body,
        out_type=jax.ShapeDtypeStruct((N, H), jnp.float32),
        mesh=mesh,
        scratch_types=[
            pltpu.VMEM((K,), jnp.int32),
            pltpu.VMEM((K,), jnp.int32),
            pltpu.VMEM((K, H), jnp.float32),
            pltpu.VMEM((SLAB, H), jnp.float32),
            pltpu.VMEM_SHARED((N, H), jnp.float32),
            pltpu.SemaphoreType.DMA,
            pltpu.SemaphoreType.DMA,
        ])(dst, zeros_feat, ones_deg)


def _tc_lin1_body(x_ref, wl_ref, wr_ref, b_ref, p_ref, r_ref):
    xb = x_ref[...]
    p_ref[...] = jnp.dot(xb, wl_ref[...], preferred_element_type=jnp.float32)
    r_ref[...] = jnp.dot(xb, wr_ref[...],
                         preferred_element_type=jnp.float32) + b_ref[...]


def _tc_lin1(x, Wl, Wr, b):
    return pl.pallas_call(
        _tc_lin1_body,
        grid=(N // ROWS_BLK,),
        in_specs=[
            pl.BlockSpec((ROWS_BLK, D), lambda i: (i, 0)),
            pl.BlockSpec((D, H), lambda i: (0, 0)),
            pl.BlockSpec((D, H), lambda i: (0, 0)),
            pl.BlockSpec((1, H), lambda i: (0, 0)),
        ],
        out_specs=[
            pl.BlockSpec((ROWS_BLK, H), lambda i: (i, 0)),
            pl.BlockSpec((ROWS_BLK, H), lambda i: (i, 0)),
        ],
        out_shape=[
            jax.ShapeDtypeStruct((N, H), jnp.float32),
            jax.ShapeDtypeStruct((N, H), jnp.float32),
        ],
    )(x, Wl, Wr, b)


def _tc_mid_body(a_ref, d_ref, r1_ref, h_ref):
    dm = jnp.maximum(d_ref[:, 0:1], 1.0)
    h_ref[...] = jnp.maximum(a_ref[...] / dm + r1_ref[...], 0.0)


def _tc_mid(a, d, R1):
    return pl.pallas_call(
        _tc_mid_body,
        grid=(N // ROWS_BLK,),
        in_specs=[
            pl.BlockSpec((ROWS_BLK, H), lambda i: (i, 0)),
            pl.BlockSpec((ROWS_BLK, H), lambda i: (i, 0)),
            pl.BlockSpec((ROWS_BLK, H), lambda i: (i, 0)),
        ],
        out_specs=pl.BlockSpec((ROWS_BLK, H), lambda i: (i, 0)),
        out_shape=jax.ShapeDtypeStruct((N, H), jnp.float32),
    )(a, d, R1)


def _tc_out_body(a_ref, d_ref, h_ref, wl_ref, wr_ref, b_ref, o_ref):
    dm = jnp.maximum(d_ref[:, 0:1], 1.0)
    mean2 = a_ref[...] / dm
    o = (jnp.dot(mean2, wl_ref[...], preferred_element_type=jnp.float32)
         + jnp.dot(h_ref[...], wr_ref[...],
                   preferred_element_type=jnp.float32)
         + b_ref[...])
    m = jnp.max(o, axis=1, keepdims=True)
    e = jnp.exp(o - m)
    lse = jnp.log(jnp.sum(e, axis=1, keepdims=True))
    o_ref[...] = o - m - lse


def _tc_out(a, d, h, Wl2, Wr2, b2):
    return pl.pallas_call(
        _tc_out_body,
        grid=(N // ROWS_BLK,),
        in_specs=[
            pl.BlockSpec((ROWS_BLK, H), lambda i: (i, 0)),
            pl.BlockSpec((ROWS_BLK, H), lambda i: (i, 0)),
            pl.BlockSpec((ROWS_BLK, H), lambda i: (i, 0)),
            pl.BlockSpec((H, C), lambda i: (0, 0)),
            pl.BlockSpec((H, C), lambda i: (0, 0)),
            pl.BlockSpec((1, C), lambda i: (0, 0)),
        ],
        out_specs=pl.BlockSpec((ROWS_BLK, C), lambda i: (i, 0)),
        out_shape=jax.ShapeDtypeStruct((N, C), jnp.float32),
    )(a, d, h, Wl2, Wr2, b2)


@jax.jit
def kernel(x, edge_index, W_l1, W_r1, b1, W_l2, W_r2, b2):
    src = edge_index[0]
    dst = edge_index[1]
    zeros_h = jnp.zeros((N, H), jnp.float32)
    ones_deg = jnp.ones((K, H), jnp.float32)

    P1, R1 = _tc_lin1(x, W_l1, W_r1, b1.reshape(1, H))
    deg = _sc_degree(dst, zeros_h, ones_deg)
    deg, P1 = lax.optimization_barrier((deg, P1))
    acc1 = _sc_aggregate(P1, src, dst, zeros_h)
    h = _tc_mid(acc1, deg, R1)
    acc2 = _sc_aggregate(h, src, dst, zeros_h)
    return _tc_out(acc2, deg, h, W_l2, W_r2, b2.reshape(1, C))


# speedup vs baseline: 1.1734x; 1.0670x over previous
import jax
import jax.numpy as jnp
from jax import lax
from jax.experimental import pallas as pl
from jax.experimental.pallas import tpu as pltpu
from jax.experimental.pallas import tpu_sc as plsc

N = 10000
E = 320000
D = 128
H = 128
C = 64

NS = 16
K = 80
RPT = 624
TAIL = N - NS * RPT
SLAB = 48
NSLAB = RPT // SLAB
DEGW = 16
ROWS_BLK = 1000


def _sc_aggregate(feat, src, dst, zeros_feat):
    mesh = plsc.VectorSubcoreMesh(core_axis_name="c", subcore_axis_name="s",
                                  num_cores=1)
    epw = E // NS

    def body(feat_hbm, src_hbm, dst_hbm, zf_hbm, acc_out, src0, src1, src2,
             dst0, dst1, dst2, rows0, rows1, rows2, stg_v, acc_sh,
             sg0, sg1, sg2, ss0, ss1, ss2):
        sid = lax.axis_index("s")
        row0 = sid * RPT

        pltpu.sync_copy(zf_hbm.at[pl.ds(0, SLAB)], stg_v)
        for j in range(NSLAB):
            pltpu.sync_copy(stg_v, acc_sh.at[pl.ds(row0 + j * SLAB, SLAB)])

        @pl.when(sid == NS - 1)
        def _():
            pltpu.sync_copy(stg_v.at[pl.ds(0, TAIL)],
                            acc_sh.at[pl.ds(NS * RPT, TAIL)])

        plsc.subcore_barrier()
        ebase = sid * epw
        srcs = (src0, src1, src2)
        dsts = (dst0, dst1, dst2)
        rows = (rows0, rows1, rows2)
        sgs = (sg0, sg1, sg2)
        sss = (ss0, ss1, ss2)

        def chunk(i, carry):
            off = ebase + i * (3 * K)
            for b in range(3):
                pltpu.sync_copy(src_hbm.at[pl.ds(off + b * K, K)], srcs[b])

                @pl.when(i > 0)
                def _():
                    pltpu.make_async_copy(rows[b], acc_sh.at[dsts[b]],
                                          sss[b]).wait()

                pltpu.async_copy(feat_hbm.at[srcs[b]], rows[b], sgs[b])
                pltpu.sync_copy(dst_hbm.at[pl.ds(off + b * K, K)], dsts[b])
            for b in range(3):
                pltpu.make_async_copy(feat_hbm.at[srcs[b]], rows[b],
                                      sgs[b]).wait()
                pltpu.async_copy(rows[b], acc_sh.at[dsts[b]], sss[b],
                                 add=True)
            return carry

        nsteps = epw // (3 * K)
        lax.fori_loop(0, nsteps, chunk, 0)
        for b in range(3):
            pltpu.make_async_copy(rows[b], acc_sh.at[dsts[b]], sss[b]).wait()
        for r in range(epw // K - 3 * nsteps):
            off = ebase + nsteps * 3 * K + r * K
            pltpu.sync_copy(src_hbm.at[pl.ds(off, K)], src0)
            pltpu.sync_copy(dst_hbm.at[pl.ds(off, K)], dst0)
            pltpu.async_copy(feat_hbm.at[src0], rows0, sg0).wait()
            pltpu.async_copy(rows0, acc_sh.at[dst0], ss0, add=True).wait()
        plsc.subcore_barrier()

        for j in range(NSLAB):
            r = row0 + j * SLAB
            pltpu.sync_copy(acc_sh.at[pl.ds(r, SLAB)], stg_v)
            pltpu.sync_copy(stg_v, acc_out.at[pl.ds(r, SLAB)])

        @pl.when(sid == NS - 1)
        def _2():
            pltpu.sync_copy(acc_sh.at[pl.ds(NS * RPT, TAIL)],
                            stg_v.at[pl.ds(0, TAIL)])
            pltpu.sync_copy(stg_v.at[pl.ds(0, TAIL)],
                            acc_out.at[pl.ds(NS * RPT, TAIL)])

    return pl.kernel(
        body,
        out_type=jax.ShapeDtypeStruct((N, H), jnp.float32),
        mesh=mesh,
        scratch_types=(
            [pltpu.VMEM((K,), jnp.int32)] * 6
            + [pltpu.VMEM((K, H), jnp.float32)] * 3
            + [pltpu.VMEM((SLAB, H), jnp.float32),
               pltpu.VMEM_SHARED((N, H), jnp.float32)]
            + [pltpu.SemaphoreType.DMA] * 6
        ))(feat, src, dst, zeros_feat)


def _sc_degree(dst, zeros_feat, ones_deg):
    mesh = plsc.VectorSubcoreMesh(core_axis_name="c", subcore_axis_name="s",
                                  num_cores=1)
    epw = E // NS

    def body(dst_hbm, zf_hbm, ones_hbm, deg_out, dst0, dst1, dst2, ones_v,
             stg_v, deg_sh, ss0, ss1, ss2):
        sid = lax.axis_index("s")
        row0 = sid * RPT

        pltpu.sync_copy(zf_hbm.at[pl.ds(0, SLAB)], stg_v)
        pltpu.sync_copy(ones_hbm, ones_v)
        for j in range(NSLAB):
            pltpu.sync_copy(stg_v, deg_sh.at[pl.ds(row0 + j * SLAB, SLAB)])

        @pl.when(sid == NS - 1)
        def _():
            pltpu.sync_copy(stg_v.at[pl.ds(0, TAIL)],
                            deg_sh.at[pl.ds(NS * RPT, TAIL)])

        plsc.subcore_barrier()
        ebase = sid * epw

        dsts = (dst0, dst1, dst2)
        sss = (ss0, ss1, ss2)

        def chunk(i, carry):
            off = ebase + i * (3 * K)
            for b in range(3):
                @pl.when(i > 0)
                def _():
                    pltpu.make_async_copy(ones_v, deg_sh.at[dsts[b]],
                                          sss[b]).wait()

                pltpu.sync_copy(dst_hbm.at[pl.ds(off + b * K, K)], dsts[b])
                pltpu.async_copy(ones_v, deg_sh.at[dsts[b]], sss[b],
                                 add=True)
            return carry

        nsteps = epw // (3 * K)
        lax.fori_loop(0, nsteps, chunk, 0)
        for b in range(3):
            pltpu.make_async_copy(ones_v, deg_sh.at[dsts[b]], sss[b]).wait()
        for r in range(epw // K - 3 * nsteps):
            off = ebase + nsteps * 3 * K + r * K
            pltpu.sync_copy(dst_hbm.at[pl.ds(off, K)], dst0)
            pltpu.async_copy(ones_v, deg_sh.at[dst0], ss0, add=True).wait()
        plsc.subcore_barrier()

        for j in range(NSLAB):
            r = row0 + j * SLAB
            pltpu.sync_copy(deg_sh.at[pl.ds(r, SLAB)], stg_v)
            pltpu.sync_copy(stg_v, deg_out.at[pl.ds(r, SLAB)])

        @pl.when(sid == NS - 1)
        def _2():
            pltpu.sync_copy(deg_sh.at[pl.ds(NS * RPT, TAIL)],
                            stg_v.at[pl.ds(0, TAIL)])
            pltpu.sync_copy(stg_v.at[pl.ds(0, TAIL)],
                            deg_out.at[pl.ds(NS * RPT, TAIL)])

    return pl.kernel(
        body,
        out_type=jax.ShapeDtypeStruct((N, H), jnp.float32),
        mesh=mesh,
        scratch_types=(
            [pltpu.VMEM((K,), jnp.int32)] * 3
            + [pltpu.VMEM((K, H), jnp.float32),
               pltpu.VMEM((SLAB, H), jnp.float32),
               pltpu.VMEM_SHARED((N, H), jnp.float32)]
            + [pltpu.SemaphoreType.DMA] * 3
        ))(dst, zeros_feat, ones_deg)


def _tc_lin1_body(x_ref, wl_ref, wr_ref, b_ref, p_ref, r_ref):
    xb = x_ref[...]
    p_ref[...] = jnp.dot(xb, wl_ref[...], preferred_element_type=jnp.float32)
    r_ref[...] = jnp.dot(xb, wr_ref[...],
                         preferred_element_type=jnp.float32) + b_ref[...]


def _tc_lin1(x, Wl, Wr, b):
    return pl.pallas_call(
        _tc_lin1_body,
        grid=(N // ROWS_BLK,),
        in_specs=[
            pl.BlockSpec((ROWS_BLK, D), lambda i: (i, 0)),
            pl.BlockSpec((D, H), lambda i: (0, 0)),
            pl.BlockSpec((D, H), lambda i: (0, 0)),
            pl.BlockSpec((1, H), lambda i: (0, 0)),
        ],
        out_specs=[
            pl.BlockSpec((ROWS_BLK, H), lambda i: (i, 0)),
            pl.BlockSpec((ROWS_BLK, H), lambda i: (i, 0)),
        ],
        out_shape=[
            jax.ShapeDtypeStruct((N, H), jnp.float32),
            jax.ShapeDtypeStruct((N, H), jnp.float32),
        ],
    )(x, Wl, Wr, b)


def _tc_mid_body(a_ref, d_ref, r1_ref, h_ref):
    dm = jnp.maximum(d_ref[:, 0:1], 1.0)
    h_ref[...] = jnp.maximum(a_ref[...] / dm + r1_ref[...], 0.0)


def _tc_mid(a, d, R1):
    return pl.pallas_call(
        _tc_mid_body,
        grid=(N // ROWS_BLK,),
        in_specs=[
            pl.BlockSpec((ROWS_BLK, H), lambda i: (i, 0)),
            pl.BlockSpec((ROWS_BLK, H), lambda i: (i, 0)),
            pl.BlockSpec((ROWS_BLK, H), lambda i: (i, 0)),
        ],
        out_specs=pl.BlockSpec((ROWS_BLK, H), lambda i: (i, 0)),
        out_shape=jax.ShapeDtypeStruct((N, H), jnp.float32),
    )(a, d, R1)


def _tc_out_body(a_ref, d_ref, h_ref, wl_ref, wr_ref, b_ref, o_ref):
    dm = jnp.maximum(d_ref[:, 0:1], 1.0)
    mean2 = a_ref[...] / dm
    o = (jnp.dot(mean2, wl_ref[...], preferred_element_type=jnp.float32)
         + jnp.dot(h_ref[...], wr_ref[...],
                   preferred_element_type=jnp.float32)
         + b_ref[...])
    m = jnp.max(o, axis=1, keepdims=True)
    e = jnp.exp(o - m)
    lse = jnp.log(jnp.sum(e, axis=1, keepdims=True))
    o_ref[...] = o - m - lse


def _tc_out(a, d, h, Wl2, Wr2, b2):
    return pl.pallas_call(
        _tc_out_body,
        grid=(N // ROWS_BLK,),
        in_specs=[
            pl.BlockSpec((ROWS_BLK, H), lambda i: (i, 0)),
            pl.BlockSpec((ROWS_BLK, H), lambda i: (i, 0)),
            pl.BlockSpec((ROWS_BLK, H), lambda i: (i, 0)),
            pl.BlockSpec((H, C), lambda i: (0, 0)),
            pl.BlockSpec((H, C), lambda i: (0, 0)),
            pl.BlockSpec((1, C), lambda i: (0, 0)),
        ],
        out_specs=pl.BlockSpec((ROWS_BLK, C), lambda i: (i, 0)),
        out_shape=jax.ShapeDtypeStruct((N, C), jnp.float32),
    )(a, d, h, Wl2, Wr2, b2)


@jax.jit
def kernel(x, edge_index, W_l1, W_r1, b1, W_l2, W_r2, b2):
    src = edge_index[0]
    dst = edge_index[1]
    zeros_h = jnp.zeros((N, H), jnp.float32)
    ones_deg = jnp.ones((K, H), jnp.float32)

    P1, R1 = _tc_lin1(x, W_l1, W_r1, b1.reshape(1, H))
    deg = _sc_degree(dst, zeros_h, ones_deg)
    deg, P1 = lax.optimization_barrier((deg, P1))
    acc1 = _sc_aggregate(P1, src, dst, zeros_h)
    h = _tc_mid(acc1, deg, R1)
    acc2 = _sc_aggregate(h, src, dst, zeros_h)
    return _tc_out(acc2, deg, h, W_l2, W_r2, b2.reshape(1, C))


# speedup vs baseline: 1.1748x; 1.0013x over previous
import jax
import jax.numpy as jnp
from jax import lax
from jax.experimental import pallas as pl
from jax.experimental.pallas import tpu as pltpu
from jax.experimental.pallas import tpu_sc as plsc

N = 10000
E = 320000
D = 128
H = 128
C = 64

NS = 16
K = 80
RPT = 624
TAIL = N - NS * RPT
SLAB = 48
NSLAB = RPT // SLAB
ROWS_BLK = 1000


def _sc_aggregate(feat, src, dst, zeros_feat):
    mesh = plsc.VectorSubcoreMesh(core_axis_name="c", subcore_axis_name="s",
                                  num_cores=1)
    epw = E // NS

    def body(feat_hbm, src_hbm, dst_hbm, zf_hbm, acc_out, src0, src1, src2,
             dst0, dst1, dst2, rows0, rows1, rows2, stg_v, acc_sh,
             sg0, sg1, sg2, ss0, ss1, ss2):
        sid = lax.axis_index("s")
        row0 = sid * RPT

        pltpu.sync_copy(zf_hbm.at[pl.ds(0, SLAB)], stg_v)
        for j in range(NSLAB):
            pltpu.sync_copy(stg_v, acc_sh.at[pl.ds(row0 + j * SLAB, SLAB)])

        @pl.when(sid == NS - 1)
        def _():
            pltpu.sync_copy(stg_v.at[pl.ds(0, TAIL)],
                            acc_sh.at[pl.ds(NS * RPT, TAIL)])

        plsc.subcore_barrier()
        ebase = sid * epw
        srcs = (src0, src1, src2)
        dsts = (dst0, dst1, dst2)
        rows = (rows0, rows1, rows2)
        sgs = (sg0, sg1, sg2)
        sss = (ss0, ss1, ss2)

        def chunk(i, carry):
            off = ebase + i * (3 * K)
            for b in range(3):
                pltpu.sync_copy(src_hbm.at[pl.ds(off + b * K, K)], srcs[b])

                @pl.when(i > 0)
                def _():
                    pltpu.make_async_copy(rows[b], acc_sh.at[dsts[b]],
                                          sss[b]).wait()

                pltpu.async_copy(feat_hbm.at[srcs[b]], rows[b], sgs[b])
                pltpu.sync_copy(dst_hbm.at[pl.ds(off + b * K, K)], dsts[b])
            for b in range(3):
                pltpu.make_async_copy(feat_hbm.at[srcs[b]], rows[b],
                                      sgs[b]).wait()
                pltpu.async_copy(rows[b], acc_sh.at[dsts[b]], sss[b],
                                 add=True)
            return carry

        nsteps = epw // (3 * K)
        lax.fori_loop(0, nsteps, chunk, 0)
        for b in range(3):
            pltpu.make_async_copy(rows[b], acc_sh.at[dsts[b]], sss[b]).wait()
        for r in range(epw // K - 3 * nsteps):
            off = ebase + nsteps * 3 * K + r * K
            pltpu.sync_copy(src_hbm.at[pl.ds(off, K)], src0)
            pltpu.sync_copy(dst_hbm.at[pl.ds(off, K)], dst0)
            pltpu.async_copy(feat_hbm.at[src0], rows0, sg0).wait()
            pltpu.async_copy(rows0, acc_sh.at[dst0], ss0, add=True).wait()
        plsc.subcore_barrier()

        for j in range(NSLAB):
            r = row0 + j * SLAB
            pltpu.sync_copy(acc_sh.at[pl.ds(r, SLAB)], stg_v)
            pltpu.sync_copy(stg_v, acc_out.at[pl.ds(r, SLAB)])

        @pl.when(sid == NS - 1)
        def _2():
            pltpu.sync_copy(acc_sh.at[pl.ds(NS * RPT, TAIL)],
                            stg_v.at[pl.ds(0, TAIL)])
            pltpu.sync_copy(stg_v.at[pl.ds(0, TAIL)],
                            acc_out.at[pl.ds(NS * RPT, TAIL)])

    return pl.kernel(
        body,
        out_type=jax.ShapeDtypeStruct((N, H), jnp.float32),
        mesh=mesh,
        scratch_types=(
            [pltpu.VMEM((K,), jnp.int32)] * 6
            + [pltpu.VMEM((K, H), jnp.float32)] * 3
            + [pltpu.VMEM((SLAB, H), jnp.float32),
               pltpu.VMEM_SHARED((N, H), jnp.float32)]
            + [pltpu.SemaphoreType.DMA] * 6
        ))(feat, src, dst, zeros_feat)


def _sc_degree(dst, zeros_feat, ones_deg):
    mesh = plsc.VectorSubcoreMesh(core_axis_name="c", subcore_axis_name="s",
                                  num_cores=1)
    epw = E // NS

    def body(dst_hbm, zf_hbm, ones_hbm, deg_out, dst0, dst1, dst2, ones_v,
             stg_v, deg_sh, ss0, ss1, ss2):
        sid = lax.axis_index("s")
        row0 = sid * RPT

        pltpu.sync_copy(zf_hbm.at[pl.ds(0, SLAB)], stg_v)
        pltpu.sync_copy(ones_hbm, ones_v)
        for j in range(NSLAB):
            pltpu.sync_copy(stg_v, deg_sh.at[pl.ds(row0 + j * SLAB, SLAB)])

        @pl.when(sid == NS - 1)
        def _():
            pltpu.sync_copy(stg_v.at[pl.ds(0, TAIL)],
                            deg_sh.at[pl.ds(NS * RPT, TAIL)])

        plsc.subcore_barrier()
        ebase = sid * epw

        dsts = (dst0, dst1, dst2)
        sss = (ss0, ss1, ss2)

        def chunk(i, carry):
            off = ebase + i * (3 * K)
            for b in range(3):
                @pl.when(i > 0)
                def _():
                    pltpu.make_async_copy(ones_v, deg_sh.at[dsts[b]],
                                          sss[b]).wait()

                pltpu.sync_copy(dst_hbm.at[pl.ds(off + b * K, K)], dsts[b])
                pltpu.async_copy(ones_v, deg_sh.at[dsts[b]], sss[b],
                                 add=True)
            return carry

        nsteps = epw // (3 * K)
        lax.fori_loop(0, nsteps, chunk, 0)
        for b in range(3):
            pltpu.make_async_copy(ones_v, deg_sh.at[dsts[b]], sss[b]).wait()
        for r in range(epw // K - 3 * nsteps):
            off = ebase + nsteps * 3 * K + r * K
            pltpu.sync_copy(dst_hbm.at[pl.ds(off, K)], dst0)
            pltpu.async_copy(ones_v, deg_sh.at[dst0], ss0, add=True).wait()
        plsc.subcore_barrier()

        for j in range(NSLAB):
            r = row0 + j * SLAB
            pltpu.sync_copy(deg_sh.at[pl.ds(r, SLAB)], stg_v)
            pltpu.sync_copy(stg_v, deg_out.at[pl.ds(r, SLAB)])

        @pl.when(sid == NS - 1)
        def _2():
            pltpu.sync_copy(deg_sh.at[pl.ds(NS * RPT, TAIL)],
                            stg_v.at[pl.ds(0, TAIL)])
            pltpu.sync_copy(stg_v.at[pl.ds(0, TAIL)],
                            deg_out.at[pl.ds(NS * RPT, TAIL)])

    return pl.kernel(
        body,
        out_type=jax.ShapeDtypeStruct((N, H), jnp.float32),
        mesh=mesh,
        scratch_types=(
            [pltpu.VMEM((K,), jnp.int32)] * 3
            + [pltpu.VMEM((K, H), jnp.float32),
               pltpu.VMEM((SLAB, H), jnp.float32),
               pltpu.VMEM_SHARED((N, H), jnp.float32)]
            + [pltpu.SemaphoreType.DMA] * 3
        ))(dst, zeros_feat, ones_deg)


def _tc_lin1_body(x_ref, wl_ref, wr_ref, b_ref, p_ref, r_ref):
    xb = x_ref[...]
    p_ref[...] = jnp.dot(xb, wl_ref[...], preferred_element_type=jnp.float32)
    r_ref[...] = jnp.dot(xb, wr_ref[...],
                         preferred_element_type=jnp.float32) + b_ref[...]


def _tc_lin1(x, Wl, Wr, b):
    return pl.pallas_call(
        _tc_lin1_body,
        grid=(N // ROWS_BLK,),
        in_specs=[
            pl.BlockSpec((ROWS_BLK, D), lambda i: (i, 0)),
            pl.BlockSpec((D, H), lambda i: (0, 0)),
            pl.BlockSpec((D, H), lambda i: (0, 0)),
            pl.BlockSpec((1, H), lambda i: (0, 0)),
        ],
        out_specs=[
            pl.BlockSpec((ROWS_BLK, H), lambda i: (i, 0)),
            pl.BlockSpec((ROWS_BLK, H), lambda i: (i, 0)),
        ],
        out_shape=[
            jax.ShapeDtypeStruct((N, H), jnp.float32),
            jax.ShapeDtypeStruct((N, H), jnp.float32),
        ],
    )(x, Wl, Wr, b)


def _tc_mid_body(a_ref, d_ref, r1_ref, h_ref):
    dm = jnp.maximum(d_ref[:, 0:1], 1.0)
    h_ref[...] = jnp.maximum(a_ref[...] / dm + r1_ref[...], 0.0)


def _tc_mid(a, d, R1):
    return pl.pallas_call(
        _tc_mid_body,
        grid=(N // ROWS_BLK,),
        in_specs=[
            pl.BlockSpec((ROWS_BLK, H), lambda i: (i, 0)),
            pl.BlockSpec((ROWS_BLK, H), lambda i: (i, 0)),
            pl.BlockSpec((ROWS_BLK, H), lambda i: (i, 0)),
        ],
        out_specs=pl.BlockSpec((ROWS_BLK, H), lambda i: (i, 0)),
        out_shape=jax.ShapeDtypeStruct((N, H), jnp.float32),
    )(a, d, R1)


def _tc_out_body(a_ref, d_ref, h_ref, wl_ref, wr_ref, b_ref, o_ref):
    dm = jnp.maximum(d_ref[:, 0:1], 1.0)
    mean2 = a_ref[...] / dm
    o = (jnp.dot(mean2, wl_ref[...], preferred_element_type=jnp.float32)
         + jnp.dot(h_ref[...], wr_ref[...],
                   preferred_element_type=jnp.float32)
         + b_ref[...])
    m = jnp.max(o, axis=1, keepdims=True)
    e = jnp.exp(o - m)
    lse = jnp.log(jnp.sum(e, axis=1, keepdims=True))
    o_ref[...] = o - m - lse


def _tc_out(a, d, h, Wl2, Wr2, b2):
    return pl.pallas_call(
        _tc_out_body,
        grid=(N // ROWS_BLK,),
        in_specs=[
            pl.BlockSpec((ROWS_BLK, H), lambda i: (i, 0)),
            pl.BlockSpec((ROWS_BLK, H), lambda i: (i, 0)),
            pl.BlockSpec((ROWS_BLK, H), lambda i: (i, 0)),
            pl.BlockSpec((H, C), lambda i: (0, 0)),
            pl.BlockSpec((H, C), lambda i: (0, 0)),
            pl.BlockSpec((1, C), lambda i: (0, 0)),
        ],
        out_specs=pl.BlockSpec((ROWS_BLK, C), lambda i: (i, 0)),
        out_shape=jax.ShapeDtypeStruct((N, C), jnp.float32),
    )(a, d, h, Wl2, Wr2, b2)


@jax.jit
def kernel(x, edge_index, W_l1, W_r1, b1, W_l2, W_r2, b2):
    src = edge_index[0]
    dst = edge_index[1]
    zeros_h = jnp.zeros((N, H), jnp.float32)
    ones_deg = jnp.ones((K, H), jnp.float32)

    P1, R1 = _tc_lin1(x, W_l1, W_r1, b1.reshape(1, H))
    deg = _sc_degree(dst, zeros_h, ones_deg)
    deg, P1 = lax.optimization_barrier((deg, P1))
    acc1 = _sc_aggregate(P1, src, dst, zeros_h)
    h = _tc_mid(acc1, deg, R1)
    acc2 = _sc_aggregate(h, src, dst, zeros_h)
    return _tc_out(acc2, deg, h, W_l2, W_r2, b2.reshape(1, C))
